# baseline (speedup 1.0000x reference)
import jax
import jax.numpy as jnp
from jax import lax
from jax.experimental import pallas as pl
from jax.experimental.pallas import tpu as pltpu
from jax.experimental.pallas import tpu_sc as plsc

D_MODEL = 128
VOCAB = 11
NC = 2
NS = 16
NW = NC * NS

ROW = 200
G1 = 128
G2 = ROW - G1
NBUF = 4
RBLK = 32
NROWS = 16384
ROWS_W = NROWS // NW
NBLK = ROWS_W // RBLK


def _sc_body(g_hbm, idx_hbm, out_hbm, table_sh, idxb0, idxb1,
             rows0, rows1, rows2, rows3,
             sem_i, sem_g, sem_o0, sem_o1, sem_o2, sem_o3):
  cid = lax.axis_index("c")
  sid = lax.axis_index("s")
  wid = sid * NC + cid
  base_row = wid * ROWS_W

  @pl.when(sid == 0)
  def _():
    pltpu.sync_copy(g_hbm, table_sh)

  plsc.subcore_barrier()

  idxbufs = (idxb0, idxb1)
  rows = (rows0, rows1, rows2, rows3)
  sem_o = (sem_o0, sem_o1, sem_o2, sem_o3)

  def wait_out(s):
    pltpu.make_async_copy(rows[s], out_hbm.at[pl.ds(0, ROW)],
                          sem_o[s]).wait()

  def g_start(idxbuf, r, s):
    pltpu.async_copy(table_sh.at[idxbuf.at[r, pl.ds(0, G1)]],
                     rows[s].at[pl.ds(0, G1)], sem_g)
    pltpu.async_copy(table_sh.at[idxbuf.at[r, pl.ds(G1, G2)]],
                     rows[s].at[pl.ds(G1, G2)], sem_g)

  def g_wait(s):
    pltpu.make_async_copy(table_sh.at[idxbufs[0].at[0, pl.ds(0, G1)]],
                          rows[s].at[pl.ds(0, G1)], sem_g).wait()
    pltpu.make_async_copy(table_sh.at[idxbufs[0].at[0, pl.ds(G1, G2)]],
                          rows[s].at[pl.ds(G1, G2)], sem_g).wait()

  def o_start(blk_out0, r, s):
    pltpu.async_copy(rows[s],
                     out_hbm.at[pl.ds(blk_out0 + r * ROW, ROW)], sem_o[s])

  def body(idxbuf, blk_out0, r, b, do_wait_out=True, lookahead=True):
    g_wait(b)
    o_start(blk_out0, r, b)
    if lookahead:
      sw = (b + 1) % NBUF
      if do_wait_out:
        wait_out(sw)
      g_start(idxbuf, r + 1, sw)

  pltpu.async_copy(idx_hbm.at[pl.ds(base_row, RBLK)], idxbufs[0], sem_i)

  for blk in range(NBLK):
    cur = blk % 2
    idxbuf = idxbufs[cur]
    blk_row0 = base_row + blk * RBLK
    blk_out0 = blk_row0 * ROW
    pltpu.make_async_copy(idx_hbm.at[pl.ds(base_row, RBLK)], idxbuf,
                          sem_i).wait()
    if blk < NBLK - 1:
      pltpu.async_copy(idx_hbm.at[pl.ds(blk_row0 + RBLK, RBLK)],
                       idxbufs[1 - cur], sem_i)

    if blk > 0:
      wait_out(0)
    g_start(idxbuf, 0, 0)

    lo = 1
    if blk == 0:
      body(idxbuf, blk_out0, 0, 0, do_wait_out=False)
      body(idxbuf, blk_out0, 1, 1, do_wait_out=False)
      body(idxbuf, blk_out0, 2, 2, do_wait_out=False)
      body(idxbuf, blk_out0, 3, 3)
    else:
      lo = 0

    @pl.loop(lo, RBLK // NBUF - 1)
    def _(k):
      for b in range(NBUF):
        body(idxbuf, blk_out0, NBUF * k + b, b)

    r0 = RBLK - NBUF
    body(idxbuf, blk_out0, r0 + 0, 0)
    body(idxbuf, blk_out0, r0 + 1, 1)
    body(idxbuf, blk_out0, r0 + 2, 2)
    body(idxbuf, blk_out0, r0 + 3, 3, lookahead=False)

  for s in range(NBUF):
    wait_out(s)


@jax.jit
def kernel(idx, G):
  b0, b1 = idx.shape
  assert b0 == NROWS and b1 == ROW
  n = b0 * b1

  mesh = plsc.VectorSubcoreMesh(core_axis_name="c", subcore_axis_name="s")
  out = pl.kernel(
      _sc_body,
      out_type=jax.ShapeDtypeStruct((n, D_MODEL), jnp.float32),
      mesh=mesh,
      scratch_types=[
          pltpu.VMEM_SHARED((VOCAB, D_MODEL), jnp.float32),
          pltpu.VMEM((RBLK, ROW), jnp.int32),
          pltpu.VMEM((RBLK, ROW), jnp.int32),
          pltpu.VMEM((ROW, D_MODEL), jnp.float32),
          pltpu.VMEM((ROW, D_MODEL), jnp.float32),
          pltpu.VMEM((ROW, D_MODEL), jnp.float32),
          pltpu.VMEM((ROW, D_MODEL), jnp.float32),
          pltpu.SemaphoreType.DMA,
          pltpu.SemaphoreType.DMA,
          pltpu.SemaphoreType.DMA,
          pltpu.SemaphoreType.DMA,
          pltpu.SemaphoreType.DMA,
          pltpu.SemaphoreType.DMA,
      ],
  )(G, idx.astype(jnp.int32))
  return out.reshape(b0, b1, D_MODEL)


# speedup vs baseline: 1.0384x; 1.0384x over previous
import jax
import jax.numpy as jnp
from jax import lax
from jax.experimental import pallas as pl
from jax.experimental.pallas import tpu as pltpu
from jax.experimental.pallas import tpu_sc as plsc

D_MODEL = 128
VOCAB = 11
NC = 2
NS = 16
NW = NC * NS

ROW = 200
G1 = 128
G2 = ROW - G1
NBUF = 4
RBLK = 32
NROWS = 16384
ROWS_W = NROWS // NW
NBLK = ROWS_W // RBLK


def _sc_body(g_hbm, idx_hbm, out_hbm, table_sh, idxb0, idxb1,
             rows0, rows1, rows2, rows3,
             sem_i, sem_g, sem_o0, sem_o1, sem_o2, sem_o3):
  cid = lax.axis_index("c")
  sid = lax.axis_index("s")
  wid = sid * NC + cid
  base_row = wid * ROWS_W

  @pl.when(sid == 0)
  def _():
    pltpu.sync_copy(g_hbm, table_sh)

  plsc.subcore_barrier()

  idxbufs = (idxb0, idxb1)
  rows = (rows0, rows1, rows2, rows3)
  sem_o = (sem_o0, sem_o1, sem_o2, sem_o3)

  def wait_out(s):
    pltpu.make_async_copy(rows[s], out_hbm.at[pl.ds(0, ROW)],
                          sem_o[s]).wait()

  def g_start(idxbuf, r, s):
    pltpu.async_copy(table_sh.at[idxbuf.at[r, pl.ds(0, G1)]],
                     rows[s].at[pl.ds(0, G1)], sem_g)
    pltpu.async_copy(table_sh.at[idxbuf.at[r, pl.ds(G1, G2)]],
                     rows[s].at[pl.ds(G1, G2)], sem_g)

  def g_wait(s):
    pltpu.make_async_copy(table_sh.at[idxbufs[0].at[0, pl.ds(0, G1)]],
                          rows[s].at[pl.ds(0, G1)], sem_g).wait()
    pltpu.make_async_copy(table_sh.at[idxbufs[0].at[0, pl.ds(G1, G2)]],
                          rows[s].at[pl.ds(G1, G2)], sem_g).wait()

  def o_start(blk_out0, r, s):
    pltpu.async_copy(rows[s],
                     out_hbm.at[pl.ds(blk_out0 + r * ROW, ROW)], sem_o[s])

  def body(idxbuf, blk_out0, r, b, do_wait_out=True, lookahead=True):
    if lookahead:
      sw = (b + 2) % NBUF
      if do_wait_out:
        wait_out(sw)
      g_start(idxbuf, r + 2, sw)
    g_wait(b)
    o_start(blk_out0, r, b)

  pltpu.async_copy(idx_hbm.at[pl.ds(base_row, RBLK)], idxbufs[0], sem_i)

  for blk in range(NBLK):
    cur = blk % 2
    idxbuf = idxbufs[cur]
    blk_row0 = base_row + blk * RBLK
    blk_out0 = blk_row0 * ROW
    pltpu.make_async_copy(idx_hbm.at[pl.ds(base_row, RBLK)], idxbuf,
                          sem_i).wait()
    if blk < NBLK - 1:
      pltpu.async_copy(idx_hbm.at[pl.ds(blk_row0 + RBLK, RBLK)],
                       idxbufs[1 - cur], sem_i)

    if blk > 0:
      wait_out(0)
    g_start(idxbuf, 0, 0)
    if blk > 0:
      wait_out(1)
    g_start(idxbuf, 1, 1)

    lo = 1
    if blk == 0:
      body(idxbuf, blk_out0, 0, 0, do_wait_out=False)
      body(idxbuf, blk_out0, 1, 1, do_wait_out=False)
      body(idxbuf, blk_out0, 2, 2)
      body(idxbuf, blk_out0, 3, 3)
    else:
      lo = 0

    @pl.loop(lo, RBLK // NBUF - 1)
    def _(k):
      for b in range(NBUF):
        body(idxbuf, blk_out0, NBUF * k + b, b)

    r0 = RBLK - NBUF
    body(idxbuf, blk_out0, r0 + 0, 0)
    body(idxbuf, blk_out0, r0 + 1, 1)
    body(idxbuf, blk_out0, r0 + 2, 2, lookahead=False)
    body(idxbuf, blk_out0, r0 + 3, 3, lookahead=False)

  for s in range(NBUF):
    wait_out(s)


@jax.jit
def kernel(idx, G):
  b0, b1 = idx.shape
  assert b0 == NROWS and b1 == ROW
  n = b0 * b1

  mesh = plsc.VectorSubcoreMesh(core_axis_name="c", subcore_axis_name="s")
  out = pl.kernel(
      _sc_body,
      out_type=jax.ShapeDtypeStruct((n, D_MODEL), jnp.float32),
      mesh=mesh,
      scratch_types=[
          pltpu.VMEM_SHARED((VOCAB, D_MODEL), jnp.float32),
          pltpu.VMEM((RBLK, ROW), jnp.int32),
          pltpu.VMEM((RBLK, ROW), jnp.int32),
          pltpu.VMEM((ROW, D_MODEL), jnp.float32),
          pltpu.VMEM((ROW, D_MODEL), jnp.float32),
          pltpu.VMEM((ROW, D_MODEL), jnp.float32),
          pltpu.VMEM((ROW, D_MODEL), jnp.float32),
          pltpu.SemaphoreType.DMA,
          pltpu.SemaphoreType.DMA,
          pltpu.SemaphoreType.DMA,
          pltpu.SemaphoreType.DMA,
          pltpu.SemaphoreType.DMA,
          pltpu.SemaphoreType.DMA,
      ],
  )(G, idx.astype(jnp.int32))
  return out.reshape(b0, b1, D_MODEL)
